# baseline (speedup 1.0000x reference)
import functools
import jax
import jax.numpy as jnp
from jax import lax
from jax.experimental import pallas as pl
from jax.experimental.pallas import tpu as pltpu
from jax.experimental.pallas import tpu_sc as plsc

NUM_CORES = 2
NUM_SUBCORES = 16
NUM_WORKERS = NUM_CORES * NUM_SUBCORES
CHUNK = 128
NBUF = 2


@functools.partial(jax.jit, static_argnums=(2,))
def _sc_gather(ids2d, table, cpw):
    n_rows = ids2d.shape[0] * CHUNK
    mesh = plsc.VectorSubcoreMesh(core_axis_name="c", subcore_axis_name="s",
                                  num_cores=NUM_CORES, num_subcores=NUM_SUBCORES)

    @functools.partial(
        pl.kernel,
        out_type=jax.ShapeDtypeStruct((n_rows, 128), jnp.float32),
        mesh=mesh,
        scratch_types=(
            [pltpu.VMEM((cpw, CHUNK), jnp.int32),
             pltpu.VMEM((2 * NBUF, CHUNK, 128), jnp.float32)]
            + [pltpu.SemaphoreType.DMA] * (4 * NBUF)),
    )
    def run(ids_hbm, table_hbm, out_hbm, idx_v, rows_v, *sems):
        gsem = sems[:2 * NBUF]
        fsem = sems[2 * NBUF:]
        wid = lax.axis_index("s") * NUM_CORES + lax.axis_index("c")
        chunk0 = wid * cpw
        pltpu.sync_copy(ids_hbm.at[pl.ds(chunk0, cpw)], idx_v)

        def gather(j, b):
            return pltpu.make_async_copy(
                table_hbm.at[idx_v.at[j]], rows_v.at[b], gsem[b])

        def flush(j, b):
            return pltpu.make_async_copy(
                rows_v.at[b],
                out_hbm.at[pl.ds((chunk0 + j) * CHUNK, CHUNK)],
                fsem[b])

        n_rounds = cpw // NBUF

        def round_ab(r, bank):
            for b in range(NBUF):
                gather(r * NBUF + b, bank * NBUF + b).wait()
            for b in range(NBUF):
                flush(r * NBUF + b, bank * NBUF + b).start()

        def round_cd(r, bank, refill):
            for b in range(NBUF):
                flush((r - 1) * NBUF + b, (1 - bank) * NBUF + b).wait()
            if refill:
                for b in range(NBUF):
                    gather((r + 1) * NBUF + b, (1 - bank) * NBUF + b).start()

        for b in range(NBUF):
            gather(b, b).start()
        for b in range(NBUF):
            gather(NBUF + b, NBUF + b).start()
        round_ab(0, 0)

        def pair_body(p, carry):
            r = 2 * p + 1
            round_ab(r, 1)
            round_cd(r, 1, True)
            round_ab(r + 1, 0)
            round_cd(r + 1, 0, True)
            return carry

        lax.fori_loop(0, (n_rounds - 4) // 2, pair_body, 0, unroll=False)
        for r in range(n_rounds - 3, n_rounds):
            round_ab(r, r % 2)
            round_cd(r, r % 2, r < n_rounds - 1)
        for b in range(NBUF):
            flush((n_rounds - 1) * NBUF + b,
                  ((n_rounds - 1) % 2) * NBUF + b).wait()

    return run(ids2d, table)


def kernel(encoder_weight, category_ids):
    batch, fields = category_ids.shape
    n = batch * fields
    span = n // NUM_WORKERS
    table128 = jnp.pad(encoder_weight, ((0, 0), (0, 64)))
    ids2d = category_ids.reshape(n // CHUNK, CHUNK)
    out = _sc_gather(ids2d, table128, span // CHUNK)
    return out[:, :64].reshape(batch, fields, 64)


# speedup vs baseline: 1.0651x; 1.0651x over previous
import functools

import jax
import jax.numpy as jnp
from jax import lax
from jax.experimental import pallas as pl
from jax.experimental.pallas import tpu as pltpu
from jax.experimental.pallas import tpu_sc as plsc

NUM_CORES = 2
NUM_SUBCORES = 16
NUM_WORKERS = NUM_CORES * NUM_SUBCORES
CHUNK = 256
NBUF = 2


@functools.partial(jax.jit, static_argnums=(2, 3))
def _sc_gather(ids2d, table, n_chunks_per_worker, embed):
    cpw = n_chunks_per_worker
    n_rows = ids2d.shape[0] * CHUNK
    mesh = plsc.VectorSubcoreMesh(
        core_axis_name="c", subcore_axis_name="s",
        num_cores=NUM_CORES, num_subcores=NUM_SUBCORES)

    @functools.partial(
        pl.kernel,
        out_type=jax.ShapeDtypeStruct((n_rows, embed), jnp.float32),
        mesh=mesh,
        scratch_types=(
            [pltpu.VMEM((cpw, CHUNK), jnp.int32),
             pltpu.VMEM((2 * NBUF, CHUNK, embed), jnp.float32)]
            + [pltpu.SemaphoreType.DMA] * (4 * NBUF)),
        compiler_params=pltpu.CompilerParams(use_tc_tiling_on_sc=False),
    )
    def run(ids_hbm, table_hbm, out_hbm, idx_v, rows_v, *sems):
        gsem = sems[:2 * NBUF]
        fsem = sems[2 * NBUF:]
        wid = lax.axis_index("s") * NUM_CORES + lax.axis_index("c")
        chunk0 = wid * cpw

        pltpu.sync_copy(ids_hbm.at[pl.ds(chunk0, cpw)], idx_v)

        def gather(j, b):
            return pltpu.make_async_copy(
                table_hbm.at[idx_v.at[j]], rows_v.at[b], gsem[b])

        def flush(j, b):
            return pltpu.make_async_copy(
                rows_v.at[b],
                out_hbm.at[pl.ds((chunk0 + j) * CHUNK, CHUNK)],
                fsem[b])

        n_rounds = cpw // NBUF
        assert n_rounds % 2 == 0 and n_rounds >= 4

        def round_ab(r, bank):
            for b in range(NBUF):
                gather(r * NBUF + b, bank * NBUF + b).wait()
            for b in range(NBUF):
                flush(r * NBUF + b, bank * NBUF + b).start()

        def round_cd(r, bank, refill):
            for b in range(NBUF):
                flush((r - 1) * NBUF + b, (1 - bank) * NBUF + b).wait()
            if refill:
                for b in range(NBUF):
                    gather((r + 1) * NBUF + b, (1 - bank) * NBUF + b).start()

        for b in range(NBUF):
            gather(b, b).start()
        for b in range(NBUF):
            gather(NBUF + b, NBUF + b).start()
        round_ab(0, 0)

        def pair_body(p, carry):
            r = 2 * p + 1
            round_ab(r, 1)
            round_cd(r, 1, True)
            round_ab(r + 1, 0)
            round_cd(r + 1, 0, True)
            return carry

        lax.fori_loop(0, (n_rounds - 4) // 2, pair_body, 0, unroll=False)
        for r in range(n_rounds - 3, n_rounds):
            round_ab(r, r % 2)
            round_cd(r, r % 2, r < n_rounds - 1)
        for b in range(NBUF):
            flush((n_rounds - 1) * NBUF + b,
                  ((n_rounds - 1) % 2) * NBUF + b).wait()

    return run(ids2d, table)


def kernel(encoder_weight, category_ids):
    batch, fields = category_ids.shape
    vocab, embed = encoder_weight.shape
    n = batch * fields
    span = n // NUM_WORKERS
    assert n % (NUM_WORKERS * CHUNK) == 0 and span % (CHUNK * NBUF) == 0
    ids2d = category_ids.reshape(n // CHUNK, CHUNK)
    out = _sc_gather(ids2d, encoder_weight, span // CHUNK, embed)
    return out.reshape(batch, fields, embed)
